# baseline (speedup 1.0000x reference)
import functools

import jax
import jax.numpy as jnp
from jax import lax
from jax.experimental import pallas as pl
from jax.experimental.pallas import tpu as pltpu
from jax.experimental.pallas import tpu_sc as plsc

B = 16384
D = 16
NC = 2
NS = 16
NW = NC * NS
BPW = B // NW

_mesh = plsc.VectorSubcoreMesh(core_axis_name="c", subcore_axis_name="s")


@functools.partial(
    pl.kernel,
    mesh=_mesh,
    out_type=jax.ShapeDtypeStruct((B,), jnp.float32),
    scratch_types=[
        pltpu.VMEM((BPW,), jnp.int32),
        pltpu.VMEM((BPW,), jnp.int32),
        pltpu.VMEM((BPW, D), jnp.float32),
        pltpu.VMEM((BPW, D), jnp.float32),
        pltpu.VMEM((BPW, D), jnp.float32),
        pltpu.VMEM((BPW, D), jnp.float32),
        pltpu.VMEM((48,), jnp.float32),
        pltpu.VMEM((16 * D,), jnp.float32),
        pltpu.VMEM((BPW,), jnp.float32),
        pltpu.SemaphoreType.DMA,
    ],
    compiler_params=pltpu.CompilerParams(
        needs_layout_passes=False, use_tc_tiling_on_sc=False
    ),
)
def _gmf_sc(u_hbm, it_hbm, u1_hbm, i1_hbm, u2_hbm, i2_hbm, wb_hbm, out_hbm,
            uidx, iidx, r1u, r1i, r2u, r2i, wv, tmpv, outv, sem):
    wid = lax.axis_index("s") * NC + lax.axis_index("c")
    base = wid * BPW

    pltpu.sync_copy(u_hbm.at[pl.ds(base, BPW)], uidx)
    pltpu.sync_copy(it_hbm.at[pl.ds(base, BPW)], iidx)
    pltpu.sync_copy(wb_hbm, wv)

    cp1 = pltpu.async_copy(u1_hbm.at[uidx], r1u, sem)
    cp2 = pltpu.async_copy(i1_hbm.at[iidx], r1i, sem)
    cp3 = pltpu.async_copy(u2_hbm.at[uidx], r2u, sem)
    cp4 = pltpu.async_copy(i2_hbm.at[iidx], r2i, sem)
    cp1.wait()
    cp2.wait()
    cp3.wait()
    cp4.wait()

    w0v = wv[pl.ds(0, 16)]
    w1v = wv[pl.ds(16, 16)]
    bv = wv[pl.ds(2 * D, 16)]
    colidx = lax.iota(jnp.int32, 16) * D

    def body(g, carry):
        gb = g * 16
        for j in range(16):
            v = (r1u[gb + j] * r1i[gb + j] * w0v
                 + r2u[gb + j] * r2i[gb + j] * w1v)
            tmpv[pl.ds(j * D, 16)] = v
        acc = bv
        for d in range(D):
            acc = acc + plsc.load_gather(tmpv, [colidx + d])
        outv[pl.ds(gb, 16)] = jnp.maximum(acc, 0.0)
        return carry

    lax.fori_loop(0, BPW // 16, body, 0)

    pltpu.sync_copy(outv, out_hbm.at[pl.ds(base, BPW)])


@jax.jit
def kernel(user_inputs, item_inputs, U1, I1, U2, I2, W, b):
    wb = jnp.concatenate([W.reshape(-1), jnp.broadcast_to(b, (16,))])
    out = _gmf_sc(user_inputs, item_inputs, U1, I1, U2, I2, wb)
    return out.reshape(B, 1)
